# baseline (speedup 1.0000x reference)
import functools

import jax
import jax.numpy as jnp
from jax.experimental import pallas as pl
from jax.experimental.pallas import tpu as pltpu

_NEG_INF = -3.0e38


def _prologue_body(x_ref, w_ref, b_ref, s2w_ref, xp_ref, s2_ref, cmax_ref):
    t = pl.program_id(0)
    xp = jax.lax.dot_general(
        x_ref[...], w_ref[...],
        dimension_numbers=(((1,), (1,)), ((), ())),
        preferred_element_type=jnp.float32,
    ) + b_ref[...]
    xp_ref[...] = xp
    s2 = jnp.sum(xp * s2w_ref[...], axis=1, keepdims=True)
    s2_ref[...] = s2

    @pl.when(t == 0)
    def _():
        cmax_ref[...] = jnp.full((1, 1), _NEG_INF, jnp.float32)

    cmax_ref[...] = jnp.maximum(cmax_ref[...],
                                jnp.max(s2, axis=(0, 1), keepdims=True))


def _vbuild_body(xp_ref, s2_ref, cmax_ref, vc_ref):
    e = jnp.exp(s2_ref[...] - cmax_ref[...])
    v = xp_ref[...] * e
    t, f = v.shape
    vc_ref[...] = jnp.concatenate(
        [v, e, jnp.zeros((t, f - 1), jnp.float32)], axis=1
    ).astype(jnp.bfloat16)


def _main_body(adj_hbm, vc_ref, vself_ref, out_ref, abuf, sems, *,
               out_f, ti, splits, nchunks):
    i = pl.program_id(0)
    tsub = ti // splits

    def _copies(chunk, buf):
        return [
            pltpu.make_async_copy(
                adj_hbm.at[pl.ds(chunk * ti + s * tsub, tsub), :],
                abuf.at[buf, pl.ds(s * tsub, tsub), :],
                sems.at[buf, s],
            )
            for s in range(splits)
        ]

    @pl.when(i == 0)
    def _():
        for c in _copies(0, 0):
            c.start()

    buf = jax.lax.rem(i, 2)
    nxt = jax.lax.rem(i + 1, 2)

    @pl.when(i + 1 < nchunks)
    def _():
        for c in _copies(i + 1, nxt):
            c.start()

    for c in _copies(i, buf):
        c.wait()

    a = pltpu.bitcast(abuf[buf], jnp.bfloat16)
    res2 = jnp.dot(a, vc_ref[...],
                   preferred_element_type=jnp.float32)
    ti_rows = res2.shape[0] // 2
    res = jnp.sum(res2.reshape(ti_rows, 2, res2.shape[1]), axis=1)
    num = res[:, :out_f] + vself_ref[:, :out_f]
    den = res[:, out_f:out_f + 1] + vself_ref[:, out_f:out_f + 1]
    out_ref[...] = jax.nn.sigmoid(num / den)


def kernel(X, adj, W, b, S):
    n, in_f = X.shape
    out_f = W.shape[0]

    tp = 1000
    ti = 400
    splits = 10
    nchunks = n // ti

    s2w = S[out_f:].reshape(1, out_f)
    b2 = b.reshape(1, out_f)

    xp, s2, cmax = pl.pallas_call(
        _prologue_body,
        grid=(n // tp,),
        in_specs=[
            pl.BlockSpec((tp, in_f), lambda t: (t, 0)),
            pl.BlockSpec((out_f, in_f), lambda t: (0, 0)),
            pl.BlockSpec((1, out_f), lambda t: (0, 0)),
            pl.BlockSpec((1, out_f), lambda t: (0, 0)),
        ],
        out_specs=[
            pl.BlockSpec((tp, out_f), lambda t: (t, 0)),
            pl.BlockSpec((tp, 1), lambda t: (t, 0)),
            pl.BlockSpec((1, 1), lambda t: (0, 0)),
        ],
        out_shape=[
            jax.ShapeDtypeStruct((n, out_f), jnp.float32),
            jax.ShapeDtypeStruct((n, 1), jnp.float32),
            jax.ShapeDtypeStruct((1, 1), jnp.float32),
        ],
    )(X, W, b2, s2w)

    vc = pl.pallas_call(
        _vbuild_body,
        grid=(n // tp,),
        in_specs=[
            pl.BlockSpec((tp, out_f), lambda t: (t, 0)),
            pl.BlockSpec((tp, 1), lambda t: (t, 0)),
            pl.BlockSpec((1, 1), lambda t: (0, 0)),
        ],
        out_specs=pl.BlockSpec((tp, 2 * out_f), lambda t: (t, 0)),
        out_shape=jax.ShapeDtypeStruct((n, 2 * out_f), jnp.bfloat16),
    )(xp, s2, cmax)

    out = pl.pallas_call(
        functools.partial(_main_body, out_f=out_f, ti=ti, splits=splits,
                          nchunks=nchunks),
        grid=(nchunks,),
        in_specs=[
            pl.BlockSpec(memory_space=pl.ANY),
            pl.BlockSpec((n, 2 * out_f), lambda i: (0, 0)),
            pl.BlockSpec((ti, 2 * out_f), lambda i: (i, 0)),
        ],
        out_specs=pl.BlockSpec((ti, out_f), lambda i: (i, 0)),
        out_shape=jax.ShapeDtypeStruct((n, out_f), jnp.float32),
        scratch_shapes=[
            pltpu.VMEM((2, ti, n), jnp.float32),
            pltpu.SemaphoreType.DMA((2, splits)),
        ],
    )(adj, vc, vc)

    return out


# speedup vs baseline: 1.4117x; 1.4117x over previous
import functools

import jax
import jax.numpy as jnp
from jax.experimental import pallas as pl
from jax.experimental.pallas import tpu as pltpu


def _body(x_ref, w_ref, b_ref, s2w_ref, adj_hbm, out_ref,
          abuf0, abuf1, sems0, sems1, vc_scr, xp_scr, s2_scr, *,
          out_f, ti, tp, splits, nchunks, n):
    i = pl.program_id(0)
    tsub = ti // splits

    def _copies(chunk, buf_ref, sem_ref):
        return [
            pltpu.make_async_copy(
                adj_hbm.at[pl.ds(chunk * ti + s * tsub, tsub), :],
                buf_ref.at[pl.ds(s * tsub, tsub), :],
                sem_ref.at[s],
            )
            for s in range(splits)
        ]

    @pl.when(i == 0)
    def _():
        for c in _copies(0, abuf0, sems0):
            c.start()

        cmax = jnp.full((1, 1), -3.0e38, jnp.float32)
        for t in range(n // tp):
            sl = pl.ds(t * tp, tp)
            xp = jax.lax.dot_general(
                x_ref[sl, :], w_ref[...],
                dimension_numbers=(((1,), (1,)), ((), ())),
                preferred_element_type=jnp.float32,
            ) + b_ref[...]
            xp_scr[sl, :] = xp
            s2 = jnp.sum(xp * s2w_ref[...], axis=1, keepdims=True)
            s2_scr[sl, :] = s2
            cmax = jnp.maximum(cmax, jnp.max(s2, axis=(0, 1), keepdims=True))

        for t in range(n // tp):
            sl = pl.ds(t * tp, tp)
            e = jnp.exp(s2_scr[sl, :] - cmax)
            v = xp_scr[sl, :] * e
            vc_scr[sl, :] = jnp.concatenate(
                [v, e, jnp.zeros((tp, out_f - 1), jnp.float32)], axis=1
            ).astype(jnp.bfloat16)

    def _step(cur_ref, cur_sems, nxt_ref, nxt_sems):
        @pl.when(i + 1 < nchunks)
        def _():
            for c in _copies(i + 1, nxt_ref, nxt_sems):
                c.start()

        for c in _copies(i, cur_ref, cur_sems):
            c.wait()

        res = jax.lax.dot_general(
            cur_ref[...], vc_scr[...],
            dimension_numbers=(((1,), (0,)), ((), ())),
            preferred_element_type=jnp.float32)
        vself = vc_scr[pl.ds(i * ti, ti), :].astype(jnp.float32)
        num = res[:, :out_f] + vself[:, :out_f]
        den = res[:, out_f:out_f + 1] + vself[:, out_f:out_f + 1]
        out_ref[...] = jax.nn.sigmoid(num / den)

    parity = jax.lax.rem(i, 2)

    @pl.when(parity == 0)
    def _():
        _step(abuf0, sems0, abuf1, sems1)

    @pl.when(parity == 1)
    def _():
        _step(abuf1, sems1, abuf0, sems0)


def kernel(X, adj, W, b, S):
    n, in_f = X.shape
    out_f = W.shape[0]

    tp = 1000
    ti = 400
    splits = 10
    nchunks = n // ti

    s2w = S[out_f:].reshape(1, out_f)
    b2 = b.reshape(1, out_f)

    out = pl.pallas_call(
        functools.partial(_body, out_f=out_f, ti=ti, tp=tp, splits=splits,
                          nchunks=nchunks, n=n),
        grid=(nchunks,),
        in_specs=[
            pl.BlockSpec((n, in_f), lambda i: (0, 0)),
            pl.BlockSpec((out_f, in_f), lambda i: (0, 0)),
            pl.BlockSpec((1, out_f), lambda i: (0, 0)),
            pl.BlockSpec((1, out_f), lambda i: (0, 0)),
            pl.BlockSpec(memory_space=pl.ANY),
        ],
        out_specs=pl.BlockSpec((ti, out_f), lambda i: (i, 0)),
        out_shape=jax.ShapeDtypeStruct((n, out_f), jnp.float32),
        scratch_shapes=[
            pltpu.VMEM((ti, n), jnp.float32),
            pltpu.VMEM((ti, n), jnp.float32),
            pltpu.SemaphoreType.DMA((splits,)),
            pltpu.SemaphoreType.DMA((splits,)),
            pltpu.VMEM((n, 2 * out_f), jnp.bfloat16),
            pltpu.VMEM((n, out_f), jnp.float32),
            pltpu.VMEM((n, 1), jnp.float32),
        ],
    )(X, W, b2, s2w, adj)

    return out


# speedup vs baseline: 1.4210x; 1.0066x over previous
import functools

import jax
import jax.numpy as jnp
from jax.experimental import pallas as pl
from jax.experimental.pallas import tpu as pltpu


def _body(x_ref, w_ref, b_ref, s2w_ref, adj_hbm, out_ref,
          abuf0, abuf1, sems0, sems1, vc_scr, xp_scr, s2_scr, *,
          out_f, ti, tp, splits, nchunks, n):
    i = pl.program_id(0)
    tsub = ti // splits

    def _copies(chunk, buf_ref, sem_ref):
        return [
            pltpu.make_async_copy(
                adj_hbm.at[pl.ds(chunk * ti + s * tsub, tsub), :],
                buf_ref.at[pl.ds(s * tsub, tsub), :],
                sem_ref.at[s],
            )
            for s in range(splits)
        ]

    @pl.when(i == 0)
    def _():
        for c in _copies(0, abuf0, sems0):
            c.start()

        cmax = jnp.full((1, 1), -3.0e38, jnp.float32)
        for t in range(n // tp):
            sl = pl.ds(t * tp, tp)
            xp = jax.lax.dot_general(
                x_ref[sl, :], w_ref[...],
                dimension_numbers=(((1,), (1,)), ((), ())),
                preferred_element_type=jnp.float32,
            ) + b_ref[...]
            xp_scr[sl, :] = xp
            s2 = jnp.sum(xp * s2w_ref[...], axis=1, keepdims=True)
            s2_scr[sl, :] = s2
            cmax = jnp.maximum(cmax, jnp.max(s2, axis=(0, 1), keepdims=True))

        for t in range(n // tp):
            sl = pl.ds(t * tp, tp)
            e = jnp.exp(s2_scr[sl, :] - cmax)
            v = xp_scr[sl, :] * e
            vc_scr[sl, :] = jnp.concatenate(
                [v, e, jnp.zeros((tp, out_f - 1), jnp.float32)], axis=1
            ).astype(jnp.bfloat16)

    def _step(cur_ref, cur_sems, nxt_ref, nxt_sems):
        @pl.when(i + 1 < nchunks)
        def _():
            for c in _copies(i + 1, nxt_ref, nxt_sems):
                c.start()

        for c in _copies(i, cur_ref, cur_sems):
            c.wait()

        res = jax.lax.dot_general(
            cur_ref[...], vc_scr[...],
            dimension_numbers=(((1,), (0,)), ((), ())),
            preferred_element_type=jnp.float32)
        vself = vc_scr[pl.ds(i * ti, ti), :].astype(jnp.float32)
        num = res[:, :out_f] + vself[:, :out_f]
        den = res[:, out_f:out_f + 1] + vself[:, out_f:out_f + 1]
        out_ref[...] = jax.nn.sigmoid(num / den)

    parity = jax.lax.rem(i, 2)

    @pl.when(parity == 0)
    def _():
        _step(abuf0, sems0, abuf1, sems1)

    @pl.when(parity == 1)
    def _():
        _step(abuf1, sems1, abuf0, sems0)


def kernel(X, adj, W, b, S):
    n, in_f = X.shape
    out_f = W.shape[0]

    tp = 1000
    ti = 400
    splits = 5
    nchunks = n // ti

    s2w = S[out_f:].reshape(1, out_f)
    b2 = b.reshape(1, out_f)

    out = pl.pallas_call(
        functools.partial(_body, out_f=out_f, ti=ti, tp=tp, splits=splits,
                          nchunks=nchunks, n=n),
        grid=(nchunks,),
        in_specs=[
            pl.BlockSpec((n, in_f), lambda i: (0, 0)),
            pl.BlockSpec((out_f, in_f), lambda i: (0, 0)),
            pl.BlockSpec((1, out_f), lambda i: (0, 0)),
            pl.BlockSpec((1, out_f), lambda i: (0, 0)),
            pl.BlockSpec(memory_space=pl.ANY),
        ],
        out_specs=pl.BlockSpec((ti, out_f), lambda i: (i, 0)),
        out_shape=jax.ShapeDtypeStruct((n, out_f), jnp.float32),
        scratch_shapes=[
            pltpu.VMEM((ti, n), jnp.float32),
            pltpu.VMEM((ti, n), jnp.float32),
            pltpu.SemaphoreType.DMA((splits,)),
            pltpu.SemaphoreType.DMA((splits,)),
            pltpu.VMEM((n, 2 * out_f), jnp.bfloat16),
            pltpu.VMEM((n, out_f), jnp.float32),
            pltpu.VMEM((n, 1), jnp.float32),
        ],
    )(X, W, b2, s2w, adj)

    return out
